# baseline (speedup 1.0000x reference)
import functools

import jax
import jax.numpy as jnp
from jax import lax
from jax.experimental import pallas as pl
from jax.experimental.pallas import tpu as pltpu
from jax.experimental.pallas import tpu_sc as plsc

BATCH = 16384
HIST = 50
EMBED_DIM = 64
VOCAB = 1000000
NB = BATCH * HIST
NW = 32
CB = 256
BPW = BATCH // NW
CPW = BPW // CB
UNITS = HIST * CPW
NBUF = 2


def _make_kernel():
  mesh = plsc.VectorSubcoreMesh(core_axis_name="c", subcore_axis_name="s")

  @functools.partial(
      pl.kernel,
      out_type=jax.ShapeDtypeStruct((HIST, EMBED_DIM, BATCH), jnp.float32),
      name="embed_gather_t",
      mesh=mesh,
      scratch_types=[
          pltpu.VMEM((CB,), jnp.int32),
          pltpu.VMEM((CB,), jnp.int32),
          pltpu.VMEM((CB, 128), jnp.float32),
          pltpu.VMEM((CB, 128), jnp.float32),
          pltpu.VMEM((EMBED_DIM, CB), jnp.float32),
          pltpu.VMEM((EMBED_DIM, CB), jnp.float32),
          pltpu.SemaphoreType.DMA((NBUF,)),
          pltpu.SemaphoreType.DMA((NBUF,)),
          pltpu.SemaphoreType.DMA((NBUF,)),
      ],
      compiler_params=pltpu.CompilerParams(
          use_tc_tiling_on_sc=True, needs_layout_passes=False),
  )
  def gather_kernel(idx_hbm, table_hbm, out_hbm, idx0, idx1, rows0, rows1,
                    outt0, outt1, idx_sem, gat_sem, st_sem):
    idx_v = [idx0, idx1]
    rows_v = [rows0, rows1]
    outt_v = [outt0, outt1]
    wid = lax.axis_index("s") * 2 + lax.axis_index("c")
    b_base = wid * BPW

    def unit_coords(u):
      h = u // CPW
      b0 = b_base + (u % CPW) * CB
      return h, b0

    def issue_idx(u, s):
      h, b0 = unit_coords(u)
      pltpu.async_copy(
          idx_hbm.at[pl.ds(h * BATCH + b0, CB)], idx_v[s], idx_sem.at[s])

    def wait_idx(u, s):
      h, b0 = unit_coords(u)
      pltpu.make_async_copy(
          idx_hbm.at[pl.ds(h * BATCH + b0, CB)], idx_v[s],
          idx_sem.at[s]).wait()

    def issue_gather(s):
      pltpu.async_copy(table_hbm.at[idx_v[s]], rows_v[s], gat_sem.at[s])

    def wait_gather(s):
      pltpu.make_async_copy(
          table_hbm.at[idx_v[s]], rows_v[s], gat_sem.at[s]).wait()

    def issue_store(u, s):
      h, b0 = unit_coords(u)
      pltpu.async_copy(
          outt_v[s], out_hbm.at[h, :, pl.ds(b0, CB)], st_sem.at[s])

    def wait_store(u, s):
      h, b0 = unit_coords(u)
      pltpu.make_async_copy(
          outt_v[s], out_hbm.at[h, :, pl.ds(b0, CB)], st_sem.at[s]).wait()

    iota = lax.iota(jnp.int32, 16)

    def transpose_unit(s):
      rows = rows_v[s]
      outt = outt_v[s]

      def dloop(d, carry):
        idx_d = jnp.full((16,), d, jnp.int32)
        for g in range(CB // 16):
          idx_b = iota + (g * 16)
          x = plsc.load_gather(rows, [idx_b, idx_d])
          outt[d, pl.ds(g * 16, 16)] = x
        return carry

      lax.fori_loop(0, EMBED_DIM, dloop, 0)

    issue_idx(0, 0)
    issue_idx(1, 1)
    wait_idx(0, 0)
    issue_gather(0)
    wait_idx(1, 1)
    issue_gather(1)

    def visit(u, s, first, last):
      wait_gather(s)
      if not first:
        wait_store(u - 2, s)
      transpose_unit(s)
      issue_store(u, s)
      if not last:
        issue_idx(u + 2, s)
        wait_idx(u + 2, s)
        issue_gather(s)

    visit(0, 0, True, False)
    visit(1, 1, True, False)

    def body(u2, carry):
      u = 2 + u2 * 2
      visit(u, 0, False, False)
      visit(u + 1, 1, False, False)
      return carry

    lax.fori_loop(0, (UNITS - 4) // 2, body, 0)

    visit(UNITS - 2, 0, False, True)
    visit(UNITS - 1, 1, False, True)
    wait_store(UNITS - 2, 0)
    wait_store(UNITS - 1, 1)

  return gather_kernel


_gather = _make_kernel()


@jax.jit
def kernel(inputs, table):
  tbl = jnp.pad(table, ((0, 0), (0, 128 - EMBED_DIM)))
  flat_idx = jnp.transpose(inputs).reshape(NB).astype(jnp.int32)
  out = _gather(flat_idx, tbl)
  return jnp.transpose(out, (2, 0, 1))


# speedup vs baseline: 1.1530x; 1.1530x over previous
import functools

import jax
import jax.numpy as jnp
from jax import lax
from jax.experimental import pallas as pl
from jax.experimental.pallas import tpu as pltpu
from jax.experimental.pallas import tpu_sc as plsc

BATCH = 16384
HIST = 50
EMBED_DIM = 64
VOCAB = 1000000
NB = BATCH * HIST
NW = 32
CB = 256
BPW = BATCH // NW
CPW = BPW // CB
UNITS = HIST * CPW
NBUF = 2


def _make_kernel():
  mesh = plsc.VectorSubcoreMesh(core_axis_name="c", subcore_axis_name="s")

  @functools.partial(
      pl.kernel,
      out_type=jax.ShapeDtypeStruct((HIST, EMBED_DIM, BATCH), jnp.float32),
      name="embed_gather_t",
      mesh=mesh,
      scratch_types=[
          pltpu.VMEM((CB,), jnp.int32),
          pltpu.VMEM((CB,), jnp.int32),
          pltpu.VMEM((CB, 128), jnp.float32),
          pltpu.VMEM((CB, 128), jnp.float32),
          pltpu.VMEM((EMBED_DIM, CB), jnp.float32),
          pltpu.VMEM((EMBED_DIM, CB), jnp.float32),
          pltpu.SemaphoreType.DMA((NBUF,)),
          pltpu.SemaphoreType.DMA((NBUF,)),
          pltpu.SemaphoreType.DMA((NBUF,)),
      ],
      compiler_params=pltpu.CompilerParams(
          use_tc_tiling_on_sc=True, needs_layout_passes=False),
  )
  def gather_kernel(idx_hbm, table_hbm, out_hbm, idx0, idx1, rows0, rows1,
                    outt0, outt1, idx_sem, gat_sem, st_sem):
    idx_v = [idx0, idx1]
    rows_v = [rows0, rows1]
    outt_v = [outt0, outt1]
    wid = lax.axis_index("s") * 2 + lax.axis_index("c")
    b_base = wid * BPW

    def unit_coords(u):
      h = u // CPW
      b0 = b_base + (u % CPW) * CB
      return h, b0

    def issue_idx(u, s):
      h, b0 = unit_coords(u)
      pltpu.async_copy(
          idx_hbm.at[pl.ds(h * BATCH + b0, CB)], idx_v[s], idx_sem.at[s])

    def wait_idx(u, s):
      h, b0 = unit_coords(u)
      pltpu.make_async_copy(
          idx_hbm.at[pl.ds(h * BATCH + b0, CB)], idx_v[s],
          idx_sem.at[s]).wait()

    def issue_gather(s):
      pltpu.async_copy(table_hbm.at[idx_v[s]], rows_v[s], gat_sem.at[s])

    def wait_gather(s):
      pltpu.make_async_copy(
          table_hbm.at[idx_v[s]], rows_v[s], gat_sem.at[s]).wait()

    def issue_store(u, s):
      h, b0 = unit_coords(u)
      pltpu.async_copy(
          outt_v[s], out_hbm.at[h, :, pl.ds(b0, CB)], st_sem.at[s])

    def wait_store(u, s):
      h, b0 = unit_coords(u)
      pltpu.make_async_copy(
          outt_v[s], out_hbm.at[h, :, pl.ds(b0, CB)], st_sem.at[s]).wait()

    iota = lax.iota(jnp.int32, 16)

    def transpose_unit(s):
      rows = rows_v[s]
      outt = outt_v[s]

      def dloop(dd, carry):
        for j in range(8):
          d = dd * 8 + j
          idx_d = jnp.full((16,), 0, jnp.int32) + d
          for g8 in range(0, CB // 16, 8):
            xs = [
                plsc.load_gather(rows, [iota + ((g8 + k) * 16), idx_d])
                for k in range(8)
            ]
            for k in range(8):
              outt[d, pl.ds((g8 + k) * 16, 16)] = xs[k]
        return carry

      lax.fori_loop(0, EMBED_DIM // 8, dloop, 0)

    issue_idx(0, 0)
    issue_idx(1, 1)
    wait_idx(0, 0)
    issue_gather(0)
    wait_idx(1, 1)
    issue_gather(1)

    def visit(u, s, first, last):
      wait_gather(s)
      if not first:
        wait_store(u - 2, s)
      transpose_unit(s)
      issue_store(u, s)
      if not last:
        issue_idx(u + 2, s)
        wait_idx(u + 2, s)
        issue_gather(s)

    visit(0, 0, True, False)
    visit(1, 1, True, False)

    def body(u2, carry):
      u = 2 + u2 * 2
      visit(u, 0, False, False)
      visit(u + 1, 1, False, False)
      return carry

    lax.fori_loop(0, (UNITS - 4) // 2, body, 0)

    visit(UNITS - 2, 0, False, True)
    visit(UNITS - 1, 1, False, True)
    wait_store(UNITS - 2, 0)
    wait_store(UNITS - 1, 1)

  return gather_kernel


_gather = _make_kernel()


@jax.jit
def kernel(inputs, table):
  tbl = jnp.pad(table, ((0, 0), (0, 128 - EMBED_DIM)))
  flat_idx = jnp.transpose(inputs).reshape(NB).astype(jnp.int32)
  out = _gather(flat_idx, tbl)
  return jnp.transpose(out, (2, 0, 1))


# speedup vs baseline: 1.1846x; 1.0273x over previous
import functools

import jax
import jax.numpy as jnp
from jax import lax
from jax.experimental import pallas as pl
from jax.experimental.pallas import tpu as pltpu
from jax.experimental.pallas import tpu_sc as plsc

BATCH = 16384
HIST = 50
EMBED_DIM = 64
VOCAB = 1000000
NB = BATCH * HIST
NW = 32
CB = 128
BPW = BATCH // NW
CPW = BPW // CB
UNITS = HIST * CPW
NROWS = 3
NOUT = 2


def _make_kernel():
  mesh = plsc.VectorSubcoreMesh(core_axis_name="c", subcore_axis_name="s")

  @functools.partial(
      pl.kernel,
      out_type=jax.ShapeDtypeStruct((HIST, EMBED_DIM, BATCH), jnp.float32),
      name="embed_gather_t",
      mesh=mesh,
      scratch_types=[
          pltpu.VMEM((CB,), jnp.int32),
          pltpu.VMEM((CB,), jnp.int32),
          pltpu.VMEM((CB,), jnp.int32),
          pltpu.VMEM((CB, 128), jnp.float32),
          pltpu.VMEM((CB, 128), jnp.float32),
          pltpu.VMEM((CB, 128), jnp.float32),
          pltpu.VMEM((EMBED_DIM, CB), jnp.float32),
          pltpu.VMEM((EMBED_DIM, CB), jnp.float32),
          pltpu.SemaphoreType.DMA((NROWS,)),
          pltpu.SemaphoreType.DMA((NROWS,)),
          pltpu.SemaphoreType.DMA((NOUT,)),
      ],
      compiler_params=pltpu.CompilerParams(
          use_tc_tiling_on_sc=True, needs_layout_passes=False),
  )
  def gather_kernel(idx_hbm, table_hbm, out_hbm, idx0, idx1, idx2, rows0,
                    rows1, rows2, outt0, outt1, idx_sem, gat_sem, st_sem):
    idx_v = [idx0, idx1, idx2]
    rows_v = [rows0, rows1, rows2]
    outt_v = [outt0, outt1]
    wid = lax.axis_index("s") * 2 + lax.axis_index("c")
    b_base = wid * BPW

    def unit_coords(u):
      h = u // CPW
      b0 = b_base + (u % CPW) * CB
      return h, b0

    def issue_idx(u, s):
      h, b0 = unit_coords(u)
      pltpu.async_copy(
          idx_hbm.at[pl.ds(h * BATCH + b0, CB)], idx_v[s], idx_sem.at[s])

    def wait_idx(u, s):
      h, b0 = unit_coords(u)
      pltpu.make_async_copy(
          idx_hbm.at[pl.ds(h * BATCH + b0, CB)], idx_v[s],
          idx_sem.at[s]).wait()

    def issue_gather(s):
      pltpu.async_copy(table_hbm.at[idx_v[s]], rows_v[s], gat_sem.at[s])

    def wait_gather(s):
      pltpu.make_async_copy(
          table_hbm.at[idx_v[s]], rows_v[s], gat_sem.at[s]).wait()

    def issue_store(u, t):
      h, b0 = unit_coords(u)
      pltpu.async_copy(
          outt_v[t], out_hbm.at[h, :, pl.ds(b0, CB)], st_sem.at[t])

    def wait_store(u, t):
      h, b0 = unit_coords(u)
      pltpu.make_async_copy(
          outt_v[t], out_hbm.at[h, :, pl.ds(b0, CB)], st_sem.at[t]).wait()

    iota = lax.iota(jnp.int32, 16)

    def transpose_unit(s, t):
      rows = rows_v[s]
      outt = outt_v[t]

      def dloop(dd, carry):
        for j in range(8):
          d = dd * 8 + j
          idx_d = jnp.full((16,), 0, jnp.int32) + d
          xs = [
              plsc.load_gather(rows, [iota + (k * 16), idx_d])
              for k in range(CB // 16)
          ]
          for k in range(CB // 16):
            outt[d, pl.ds(k * 16, 16)] = xs[k]
        return carry

      lax.fori_loop(0, EMBED_DIM // 8, dloop, 0)

    def visit(u, s, t, first, last, refill=True):
      wait_gather(s)
      if not last:
        if refill:
          issue_idx(u + NROWS, s)
        wait_idx(u + 2, (s + 2) % NROWS)
        issue_gather((s + 2) % NROWS)
      if not first:
        wait_store(u - NOUT, t)
      transpose_unit(s, t)
      issue_store(u, t)

    issue_idx(0, 0)
    issue_idx(1, 1)
    issue_idx(2, 2)
    wait_idx(0, 0)
    issue_gather(0)
    wait_idx(1, 1)
    issue_gather(1)

    visit(0, 0, 0, True, False)
    visit(1, 1, 1, True, False)

    def body(g, carry):
      for j in range(6):
        visit(2 + g * 6 + j, (2 + j) % NROWS, j % NOUT, False, False)
      return carry

    lax.fori_loop(0, (UNITS - 8) // 6, body, 0)

    for u in range(UNITS - 6, UNITS - 3):
      visit(u, u % NROWS, u % NOUT, False, False)
    visit(UNITS - 3, (UNITS - 3) % NROWS, (UNITS - 3) % NOUT, False, False,
          refill=False)
    visit(UNITS - 2, (UNITS - 2) % NROWS, (UNITS - 2) % NOUT, False, True)
    visit(UNITS - 1, (UNITS - 1) % NROWS, (UNITS - 1) % NOUT, False, True)
    wait_store(UNITS - 2, (UNITS - 2) % NOUT)
    wait_store(UNITS - 1, (UNITS - 1) % NOUT)

  return gather_kernel


_gather = _make_kernel()


@jax.jit
def kernel(inputs, table):
  tbl = jnp.pad(table, ((0, 0), (0, 128 - EMBED_DIM)))
  flat_idx = jnp.transpose(inputs).reshape(NB).astype(jnp.int32)
  out = _gather(flat_idx, tbl)
  return jnp.transpose(out, (2, 0, 1))
